# baseline (speedup 1.0000x reference)
import functools

import jax
import jax.numpy as jnp
from jax import lax
from jax.experimental import pallas as pl
from jax.experimental.pallas import tpu as pltpu
from jax.experimental.pallas import tpu_sc as plsc

_LANES = 128
_GROUP = 8
_NBUF = 4


@functools.lru_cache(maxsize=None)
def _build(batch, hist, vocab, dim):
  info = plsc.get_sparse_core_info()
  nc, ns = info.num_cores, info.num_subcores
  nw = nc * ns
  assert batch % (nw * _GROUP * _NBUF) == 0
  rows_per_w = batch // nw
  n_groups = rows_per_w // _GROUP
  n_rounds = n_groups // _NBUF

  mesh = plsc.VectorSubcoreMesh(core_axis_name="c", subcore_axis_name="s")
  hist_p = (hist + 7) // 8 * 8

  @functools.partial(
      pl.kernel,
      out_type=jax.ShapeDtypeStruct((batch, hist, dim), jnp.float32),
      mesh=mesh,
      scratch_types=[
          pltpu.VMEM((rows_per_w, hist_p), jnp.int32),
          pltpu.VMEM((_NBUF, _GROUP, hist_p, dim), jnp.float32),
          pltpu.SemaphoreType.DMA((_NBUF,)),
          pltpu.SemaphoreType.DMA((_NBUF,)),
      ],
      compiler_params=pltpu.CompilerParams(use_tc_tiling_on_sc=False),
  )
  def k(idx_hbm, table_hbm, out_hbm, idx_v, rows_v, gsem, osem):
    wid = lax.axis_index("s") * nc + lax.axis_index("c")
    base = wid * rows_per_w
    pltpu.sync_copy(
        idx_hbm.at[pl.ds(base, rows_per_w), pl.ds(0, hist_p)], idx_v
    )

    def gather(g, b):
      for i in range(_GROUP):
        pltpu.async_copy(
            table_hbm.at[idx_v.at[g * _GROUP + i]],
            rows_v.at[b, i],
            gsem.at[b],
        )

    def wb_start(g, b):
      for i in range(_GROUP):
        pltpu.sync_copy(
            rows_v.at[b, i, pl.ds(0, hist)], out_hbm.at[base + g * _GROUP + i]
        )

    def wait(sem, b):
      for i in range(_GROUP):
        pltpu.make_async_copy(
            table_hbm.at[pl.ds(0, hist_p)], rows_v.at[b, i], sem.at[b]
        ).wait()

    for b in range(_NBUF):
      gather(b, b)

    def body(r, carry):
      g0 = r * _NBUF
      for b in range(_NBUF):
        g = g0 + b
        wait(gsem, b)
        wb_start(g, b)
        gather(g + _NBUF, b)
      return carry

    lax.fori_loop(0, n_rounds - 1, body, 0)

    g0 = (n_rounds - 1) * _NBUF
    for b in range(_NBUF):
      wait(gsem, b)
      wb_start(g0 + b, b)

  return k


def kernel(input, table):
  batch, hist = input.shape
  vocab, dim = table.shape
  idx_pad = jnp.pad(input.astype(jnp.int32), ((0, 0), (0, _LANES - hist)))
  return _build(batch, hist, vocab, dim)(idx_pad, table)


# speedup vs baseline: 1.0045x; 1.0045x over previous
import functools

import jax
import jax.numpy as jnp
from jax import lax
from jax.experimental import pallas as pl
from jax.experimental.pallas import tpu as pltpu
from jax.experimental.pallas import tpu_sc as plsc

_LANES = 128
_GROUP = 8
_NBUF = 4


@functools.lru_cache(maxsize=None)
def _build(batch, hist, vocab, dim):
  info = plsc.get_sparse_core_info()
  nc, ns = info.num_cores, info.num_subcores
  nw = nc * ns
  assert batch % (nw * _GROUP * _NBUF) == 0
  rows_per_w = batch // nw
  n_groups = rows_per_w // _GROUP
  hist_p = (hist + 7) // 8 * 8
  slot = _GROUP * hist_p

  mesh = plsc.VectorSubcoreMesh(core_axis_name="c", subcore_axis_name="s")

  @functools.partial(
      pl.kernel,
      out_type=jax.ShapeDtypeStruct((batch, hist, dim), jnp.float32),
      mesh=mesh,
      scratch_types=[
          pltpu.VMEM((rows_per_w, hist_p), jnp.int32),
          pltpu.VMEM((_NBUF, slot, dim), jnp.float32),
          pltpu.SemaphoreType.DMA((_NBUF,)),
          pltpu.SemaphoreType.DMA((_NBUF,)),
      ],
      compiler_params=pltpu.CompilerParams(use_tc_tiling_on_sc=False),
  )
  def k(idx_hbm, table_hbm, out_hbm, idx_v, rows_v, gsem, osem):
    wid = lax.axis_index("s") * nc + lax.axis_index("c")
    base = wid * rows_per_w
    pltpu.sync_copy(
        idx_hbm.at[pl.ds(base, rows_per_w), pl.ds(0, hist_p)], idx_v
    )

    def gathers(g, s):
      for i in range(_GROUP):
        pltpu.async_copy(
            table_hbm.at[idx_v.at[g * _GROUP + i]],
            rows_v.at[s, pl.ds(i * hist_p, hist_p)],
            gsem.at[s],
        )

    def writes(g, s):
      for i in range(_GROUP):
        pltpu.async_copy(
            rows_v.at[s, pl.ds(i * hist_p, hist)],
            out_hbm.at[base + g * _GROUP + i],
            osem.at[s],
        )

    def wait_gathers(s):
      pltpu.make_async_copy(
          table_hbm.at[pl.ds(0, slot)], rows_v.at[s], gsem.at[s]
      ).wait()

    def wait_writes(s):
      pltpu.make_async_copy(
          table_hbm.at[pl.ds(0, _GROUP * hist)],
          rows_v.at[s, pl.ds(0, _GROUP * hist)],
          osem.at[s],
      ).wait()

    for s in range(_NBUF):
      gathers(s, s)

    def body(g, carry):
      s = lax.rem(g, _NBUF)
      wait_gathers(s)
      writes(g, s)
      wait_writes(s)

      @pl.when(g + _NBUF < n_groups)
      def _():
        gathers(g + _NBUF, s)

      return carry

    lax.fori_loop(0, n_groups, body, 0)

  return k


def kernel(input, table):
  batch, hist = input.shape
  vocab, dim = table.shape
  idx_pad = jnp.pad(input.astype(jnp.int32), ((0, 0), (0, _LANES - hist)))
  return _build(batch, hist, vocab, dim)(idx_pad, table)
